# baseline (speedup 1.0000x reference)
import functools

import jax
import jax.numpy as jnp
from jax import lax
from jax.experimental import pallas as pl
from jax.experimental.pallas import tpu as pltpu
from jax.experimental.pallas import tpu_sc as plsc

L = 16
NC, NS = 2, 16
NW = NC * NS
D = 64
BAG = 8
C = 16
IPC = C * BAG
ROWS_PER_DMA = min(IPC, 128)
NDMA = IPC // ROWS_PER_DMA

B_POI = 1024 * 20
B_NB = 1024 * 20 * 8
TOTAL_BAGS = B_POI + B_NB
BAGS_PER_W = TOTAL_BAGS // NW
NCHUNKS = BAGS_PER_W // C
POI_GCHUNKS = B_POI // C
DEPTH = 3


def _sc_body(emb_hbm, poi_hbm, nb_hbm, opoi_hbm, onb_hbm, *scratch):
    wid = lax.axis_index("s") * NC + lax.axis_index("c")
    idx = scratch[0:DEPTH]
    rows = scratch[DEPTH:2 * DEPTH]
    outb = scratch[2 * DEPTH:3 * DEPTH]
    semg = scratch[3 * DEPTH:4 * DEPTH]
    semi = scratch[4 * DEPTH:5 * DEPTH]
    semo = scratch[5 * DEPTH:6 * DEPTH]
    lanes_hi = lax.iota(jnp.int32, L) >= BAG

    def fire_ids(cc, par):
        g = wid * NCHUNKS + cc

        @pl.when(g < POI_GCHUNKS)
        def _():
            pltpu.async_copy(poi_hbm.at[pl.ds(g * IPC, IPC)], idx[par],
                             semi[par])

        @pl.when(g >= POI_GCHUNKS)
        def _():
            pltpu.async_copy(nb_hbm.at[pl.ds((g - POI_GCHUNKS) * IPC, IPC)],
                             idx[par], semi[par])

    def wait_ids(par):
        pltpu.make_async_copy(poi_hbm.at[pl.ds(0, IPC)], idx[par],
                              semi[par]).wait()

    def fire_gathers(par):
        for i in range(NDMA):
            pltpu.async_copy(
                emb_hbm.at[idx[par].at[pl.ds(i * ROWS_PER_DMA,
                                             ROWS_PER_DMA)]],
                rows[par].at[pl.ds(i * ROWS_PER_DMA, ROWS_PER_DMA), :],
                semg[par])

    def wait_gathers(par):
        for i in range(NDMA):
            pltpu.make_async_copy(
                emb_hbm.at[idx[par].at[pl.ds(i * ROWS_PER_DMA,
                                             ROWS_PER_DMA)]],
                rows[par].at[pl.ds(i * ROWS_PER_DMA, ROWS_PER_DMA), :],
                semg[par]).wait()

    def fire_out(cc, par):
        g = wid * NCHUNKS + cc

        @pl.when(g < POI_GCHUNKS)
        def _():
            pltpu.async_copy(outb[par], opoi_hbm.at[pl.ds(g * C * D, C * D)],
                             semo[par])

        @pl.when(g >= POI_GCHUNKS)
        def _():
            pltpu.async_copy(
                outb[par],
                onb_hbm.at[pl.ds((g - POI_GCHUNKS) * C * D, C * D)],
                semo[par])

    def wait_out(par):
        pltpu.make_async_copy(outb[par], opoi_hbm.at[pl.ds(0, C * D)],
                              semo[par]).wait()

    def compute(par):
        for t in range(C // 2):
            v = idx[par][pl.ds(t * L, L)]
            m = jnp.where(v != 0, 1.0, 0.0)
            cs = plsc.cumsum(m)
            c0 = jnp.broadcast_to(cs[BAG - 1], (L,))
            cnts = cs - jnp.where(lanes_hi, c0, 0.0)
            recv = 1.0 / jnp.maximum(cnts, 1.0)
            recs = (jnp.broadcast_to(recv[BAG - 1], (L,)),
                    jnp.broadcast_to(recv[2 * BAG - 1], (L,)))
            for h in range(2):
                b = t * 2 + h
                rbase = b * BAG
                for k in range(D // L):
                    acc = rows[par][rbase, pl.ds(k * L, L)]
                    for j in range(1, BAG):
                        acc = acc + rows[par][rbase + j, pl.ds(k * L, L)]
                    outb[par][pl.ds(b * D + k * L, L)] = acc * recs[h]

    for s in range(DEPTH):
        fire_ids(s, s)
    for s in range(DEPTH - 1):
        wait_ids(s)
        fire_gathers(s)

    def step(c2, carry):
        for par in range(DEPTH):
            c = c2 * DEPTH + par
            wait_gathers(par)

            @pl.when(c + DEPTH - 1 < NCHUNKS)
            def _():
                wait_ids((par + DEPTH - 1) % DEPTH)
                fire_gathers((par + DEPTH - 1) % DEPTH)

            @pl.when(c >= DEPTH)
            def _():
                wait_out(par)

            compute(par)

            @pl.when(c + DEPTH < NCHUNKS)
            def _():
                fire_ids(c + DEPTH, par)

            fire_out(c, par)
        return carry

    lax.fori_loop(0, NCHUNKS // DEPTH, step, 0)
    for s in range(DEPTH):
        wait_out(s)


@functools.partial(jax.jit, static_argnames=())
def _sc_pool(emb, poi_flat, nb_flat):
    kfn = pl.kernel(
        _sc_body,
        out_type=(jax.ShapeDtypeStruct((B_POI * D,), jnp.float32),
                  jax.ShapeDtypeStruct((B_NB * D,), jnp.float32)),
        mesh=plsc.VectorSubcoreMesh(core_axis_name="c", subcore_axis_name="s"),
        scratch_types=(
            [pltpu.VMEM((IPC,), jnp.int32) for _ in range(DEPTH)]
            + [pltpu.VMEM((IPC, D), jnp.float32) for _ in range(DEPTH)]
            + [pltpu.VMEM((C * D,), jnp.float32) for _ in range(DEPTH)]
            + [pltpu.SemaphoreType.DMA for _ in range(3 * DEPTH)]
        ),
        compiler_params=pltpu.CompilerParams(use_tc_tiling_on_sc=False,
                                             needs_layout_passes=False),
    )
    return kfn(emb, poi_flat, nb_flat)


def kernel(poi_ids, neighbor_ids, embedding):
    poi_flat = poi_ids.reshape(-1).astype(jnp.int32)
    nb_flat = neighbor_ids.reshape(-1).astype(jnp.int32)
    opoi, onb = _sc_pool(embedding, poi_flat, nb_flat)
    return (opoi.reshape(1024, 20, D), onb.reshape(1024, 20, BAG, D))


# speedup vs baseline: 1.1721x; 1.1721x over previous
import functools

import jax
import jax.numpy as jnp
from jax import lax
from jax.experimental import pallas as pl
from jax.experimental.pallas import tpu as pltpu
from jax.experimental.pallas import tpu_sc as plsc

L = 16
NC, NS = 2, 16
NW = NC * NS
D = 64
BAG = 8
C = 8
IPC = C * BAG
ROWS_PER_DMA = min(IPC, 128)
NDMA = IPC // ROWS_PER_DMA

B_POI = 1024 * 20
B_NB = 1024 * 20 * 8
TOTAL_BAGS = B_POI + B_NB
BAGS_PER_W = TOTAL_BAGS // NW
NCHUNKS = BAGS_PER_W // C
POI_GCHUNKS = B_POI // C
DEPTH = 3


def _sc_body(emb_hbm, poi_hbm, nb_hbm, opoi_hbm, onb_hbm, *scratch):
    wid = lax.axis_index("s") * NC + lax.axis_index("c")
    idx = scratch[0:DEPTH]
    rows = scratch[DEPTH:2 * DEPTH]
    outb = scratch[2 * DEPTH:3 * DEPTH]
    semg = scratch[3 * DEPTH:4 * DEPTH]
    semi = scratch[4 * DEPTH:5 * DEPTH]
    semo = scratch[5 * DEPTH:6 * DEPTH]
    lanes_hi = lax.iota(jnp.int32, L) >= BAG

    def fire_ids(cc, par):
        g = wid * NCHUNKS + cc

        @pl.when(g < POI_GCHUNKS)
        def _():
            pltpu.async_copy(poi_hbm.at[pl.ds(g * IPC, IPC)], idx[par],
                             semi[par])

        @pl.when(g >= POI_GCHUNKS)
        def _():
            pltpu.async_copy(nb_hbm.at[pl.ds((g - POI_GCHUNKS) * IPC, IPC)],
                             idx[par], semi[par])

    def wait_ids(par):
        pltpu.make_async_copy(poi_hbm.at[pl.ds(0, IPC)], idx[par],
                              semi[par]).wait()

    def fire_gathers(par):
        for i in range(NDMA):
            pltpu.async_copy(
                emb_hbm.at[idx[par].at[pl.ds(i * ROWS_PER_DMA,
                                             ROWS_PER_DMA)]],
                rows[par].at[pl.ds(i * ROWS_PER_DMA, ROWS_PER_DMA), :],
                semg[par])

    def wait_gathers(par):
        for i in range(NDMA):
            pltpu.make_async_copy(
                emb_hbm.at[idx[par].at[pl.ds(i * ROWS_PER_DMA,
                                             ROWS_PER_DMA)]],
                rows[par].at[pl.ds(i * ROWS_PER_DMA, ROWS_PER_DMA), :],
                semg[par]).wait()

    def fire_out(cc, par):
        g = wid * NCHUNKS + cc

        @pl.when(g < POI_GCHUNKS)
        def _():
            pltpu.async_copy(outb[par], opoi_hbm.at[pl.ds(g * C * D, C * D)],
                             semo[par])

        @pl.when(g >= POI_GCHUNKS)
        def _():
            pltpu.async_copy(
                outb[par],
                onb_hbm.at[pl.ds((g - POI_GCHUNKS) * C * D, C * D)],
                semo[par])

    def wait_out(par):
        pltpu.make_async_copy(outb[par], opoi_hbm.at[pl.ds(0, C * D)],
                              semo[par]).wait()

    def compute(par):
        for t in range(C // 2):
            v = idx[par][pl.ds(t * L, L)]
            m = jnp.where(v != 0, 1.0, 0.0)
            cs = plsc.cumsum(m)
            c0 = jnp.broadcast_to(cs[BAG - 1], (L,))
            cnts = cs - jnp.where(lanes_hi, c0, 0.0)
            recv = 1.0 / jnp.maximum(cnts, 1.0)
            recs = (jnp.broadcast_to(recv[BAG - 1], (L,)),
                    jnp.broadcast_to(recv[2 * BAG - 1], (L,)))
            for h in range(2):
                b = t * 2 + h
                rbase = b * BAG
                for k in range(D // L):
                    acc = rows[par][rbase, pl.ds(k * L, L)]
                    for j in range(1, BAG):
                        acc = acc + rows[par][rbase + j, pl.ds(k * L, L)]
                    outb[par][pl.ds(b * D + k * L, L)] = acc * recs[h]

    for s in range(DEPTH):
        fire_ids(s, s)
    for s in range(DEPTH - 1):
        wait_ids(s)
        fire_gathers(s)

    def step(c2, carry):
        for par in range(DEPTH):
            c = c2 * DEPTH + par
            wait_gathers(par)

            @pl.when(c + DEPTH - 1 < NCHUNKS)
            def _():
                wait_ids((par + DEPTH - 1) % DEPTH)
                fire_gathers((par + DEPTH - 1) % DEPTH)

            @pl.when(c >= DEPTH)
            def _():
                wait_out(par)

            compute(par)

            @pl.when(c + DEPTH < NCHUNKS)
            def _():
                fire_ids(c + DEPTH, par)

            fire_out(c, par)
        return carry

    lax.fori_loop(0, NCHUNKS // DEPTH, step, 0)
    for s in range(DEPTH):
        wait_out(s)


@functools.partial(jax.jit, static_argnames=())
def _sc_pool(emb, poi_flat, nb_flat):
    kfn = pl.kernel(
        _sc_body,
        out_type=(jax.ShapeDtypeStruct((B_POI * D,), jnp.float32),
                  jax.ShapeDtypeStruct((B_NB * D,), jnp.float32)),
        mesh=plsc.VectorSubcoreMesh(core_axis_name="c", subcore_axis_name="s"),
        scratch_types=(
            [pltpu.VMEM((IPC,), jnp.int32) for _ in range(DEPTH)]
            + [pltpu.VMEM((IPC, D), jnp.float32) for _ in range(DEPTH)]
            + [pltpu.VMEM((C * D,), jnp.float32) for _ in range(DEPTH)]
            + [pltpu.SemaphoreType.DMA for _ in range(3 * DEPTH)]
        ),
        compiler_params=pltpu.CompilerParams(use_tc_tiling_on_sc=False,
                                             needs_layout_passes=False),
    )
    return kfn(emb, poi_flat, nb_flat)


def kernel(poi_ids, neighbor_ids, embedding):
    poi_flat = poi_ids.reshape(-1).astype(jnp.int32)
    nb_flat = neighbor_ids.reshape(-1).astype(jnp.int32)
    opoi, onb = _sc_pool(embedding, poi_flat, nb_flat)
    return (opoi.reshape(1024, 20, D), onb.reshape(1024, 20, BAG, D))


# speedup vs baseline: 1.3051x; 1.1134x over previous
import functools

import jax
import jax.numpy as jnp
from jax import lax
from jax.experimental import pallas as pl
from jax.experimental.pallas import tpu as pltpu
from jax.experimental.pallas import tpu_sc as plsc

L = 16
NC, NS = 2, 16
NW = NC * NS
D = 64
BAG = 8
C = 8
IPC = C * BAG
ROWS_PER_DMA = min(IPC, 128)
NDMA = IPC // ROWS_PER_DMA

B_POI = 1024 * 20
B_NB = 1024 * 20 * 8
TOTAL_BAGS = B_POI + B_NB
BAGS_PER_W = TOTAL_BAGS // NW
NCHUNKS = BAGS_PER_W // C
POI_GCHUNKS = B_POI // C
DEPTH = 3


def _sc_body(emb_hbm, poi_hbm, nb_hbm, opoi_hbm, onb_hbm, *scratch):
    wid = lax.axis_index("s") * NC + lax.axis_index("c")
    idx = scratch[0:DEPTH]
    rows = scratch[DEPTH:2 * DEPTH]
    outb = scratch[2 * DEPTH:3 * DEPTH]
    semg = scratch[3 * DEPTH:4 * DEPTH]
    semi = scratch[4 * DEPTH:5 * DEPTH]
    semo = scratch[5 * DEPTH:6 * DEPTH]
    lanes_hi = lax.iota(jnp.int32, L) >= BAG

    def fire_ids(cc, par):
        g = wid * NCHUNKS + cc

        @pl.when(g < POI_GCHUNKS)
        def _():
            pltpu.async_copy(poi_hbm.at[pl.ds(g * IPC, IPC)], idx[par],
                             semi[par])

        @pl.when(g >= POI_GCHUNKS)
        def _():
            pltpu.async_copy(nb_hbm.at[pl.ds((g - POI_GCHUNKS) * IPC, IPC)],
                             idx[par], semi[par])

    def wait_ids(par):
        pltpu.make_async_copy(poi_hbm.at[pl.ds(0, IPC)], idx[par],
                              semi[par]).wait()

    def fire_gathers(par):
        for i in range(NDMA):
            pltpu.async_copy(
                emb_hbm.at[idx[par].at[pl.ds(i * ROWS_PER_DMA,
                                             ROWS_PER_DMA)]],
                rows[par].at[pl.ds(i * ROWS_PER_DMA, ROWS_PER_DMA), :],
                semg[par])

    def wait_gathers(par):
        for i in range(NDMA):
            pltpu.make_async_copy(
                emb_hbm.at[idx[par].at[pl.ds(i * ROWS_PER_DMA,
                                             ROWS_PER_DMA)]],
                rows[par].at[pl.ds(i * ROWS_PER_DMA, ROWS_PER_DMA), :],
                semg[par]).wait()

    def fire_out(cc, par):
        g = wid * NCHUNKS + cc

        @pl.when(g < POI_GCHUNKS)
        def _():
            pltpu.async_copy(outb[par], opoi_hbm.at[pl.ds(g * C * D, C * D)],
                             semo[par])

        @pl.when(g >= POI_GCHUNKS)
        def _():
            pltpu.async_copy(
                outb[par],
                onb_hbm.at[pl.ds((g - POI_GCHUNKS) * C * D, C * D)],
                semo[par])

    def wait_out(par):
        pltpu.make_async_copy(outb[par], opoi_hbm.at[pl.ds(0, C * D)],
                              semo[par]).wait()

    hi_mask = jnp.int32(-65536)

    def compute(par):
        for t in range(C // 2):
            v = idx[par][pl.ds(t * L, L)]
            m = jnp.where(v != 0, 1.0, 0.0)
            cs = plsc.cumsum(m)
            c0 = jnp.broadcast_to(cs[BAG - 1], (L,))
            cnts = cs - jnp.where(lanes_hi, c0, 0.0)
            recv = 1.0 / jnp.maximum(cnts, 1.0)
            recs = (jnp.broadcast_to(recv[BAG - 1], (L,)),
                    jnp.broadcast_to(recv[2 * BAG - 1], (L,)))
            for h in range(2):
                b = t * 2 + h
                rbase = b * BAG
                for k in range(2):
                    x = plsc.bitcast(rows[par][rbase, pl.ds(k * 32, 32)],
                                     jnp.int32)
                    acc_lo = lax.shift_left(x, 16)
                    acc_hi = jnp.bitwise_and(x, hi_mask)
                    acc_lo = plsc.bitcast(acc_lo, jnp.float32)
                    acc_hi = plsc.bitcast(acc_hi, jnp.float32)
                    for j in range(1, BAG):
                        x = plsc.bitcast(
                            rows[par][rbase + j, pl.ds(k * 32, 32)],
                            jnp.int32)
                        acc_lo = acc_lo + plsc.bitcast(
                            lax.shift_left(x, 16), jnp.float32)
                        acc_hi = acc_hi + plsc.bitcast(
                            jnp.bitwise_and(x, hi_mask), jnp.float32)
                    outb[par][pl.ds(b * D + k * 32, L)] = acc_lo * recs[h]
                    outb[par][pl.ds(b * D + k * 32 + L, L)] = (
                        acc_hi * recs[h])

    for s in range(DEPTH):
        fire_ids(s, s)
    for s in range(DEPTH - 1):
        wait_ids(s)
        fire_gathers(s)

    def step(c2, carry):
        for par in range(DEPTH):
            c = c2 * DEPTH + par
            wait_gathers(par)

            @pl.when(c + DEPTH - 1 < NCHUNKS)
            def _():
                wait_ids((par + DEPTH - 1) % DEPTH)
                fire_gathers((par + DEPTH - 1) % DEPTH)

            @pl.when(c >= DEPTH)
            def _():
                wait_out(par)

            compute(par)

            @pl.when(c + DEPTH < NCHUNKS)
            def _():
                fire_ids(c + DEPTH, par)

            fire_out(c, par)
        return carry

    lax.fori_loop(0, NCHUNKS // DEPTH, step, 0)
    for s in range(DEPTH):
        wait_out(s)


@functools.partial(jax.jit, static_argnames=())
def _sc_pool(emb, poi_flat, nb_flat):
    kfn = pl.kernel(
        _sc_body,
        out_type=(jax.ShapeDtypeStruct((B_POI * D,), jnp.float32),
                  jax.ShapeDtypeStruct((B_NB * D,), jnp.float32)),
        mesh=plsc.VectorSubcoreMesh(core_axis_name="c", subcore_axis_name="s"),
        scratch_types=(
            [pltpu.VMEM((IPC,), jnp.int32) for _ in range(DEPTH)]
            + [pltpu.VMEM((IPC, D), jnp.bfloat16) for _ in range(DEPTH)]
            + [pltpu.VMEM((C * D,), jnp.float32) for _ in range(DEPTH)]
            + [pltpu.SemaphoreType.DMA for _ in range(3 * DEPTH)]
        ),
        compiler_params=pltpu.CompilerParams(use_tc_tiling_on_sc=False,
                                             needs_layout_passes=False),
    )
    return kfn(emb, poi_flat, nb_flat)


_COL_PERM = [blk * 32 + off
             for blk in range(2)
             for i in range(16)
             for off in (i, 16 + i)]


def kernel(poi_ids, neighbor_ids, embedding):
    poi_flat = poi_ids.reshape(-1).astype(jnp.int32)
    nb_flat = neighbor_ids.reshape(-1).astype(jnp.int32)
    emb_bf16 = embedding[:, jnp.array(_COL_PERM)].astype(jnp.bfloat16)
    opoi, onb = _sc_pool(emb_bf16, poi_flat, nb_flat)
    return (opoi.reshape(1024, 20, D), onb.reshape(1024, 20, BAG, D))


# speedup vs baseline: 1.6000x; 1.2260x over previous
import functools

import jax
import jax.numpy as jnp
from jax import lax
from jax.experimental import pallas as pl
from jax.experimental.pallas import tpu as pltpu
from jax.experimental.pallas import tpu_sc as plsc

L = 16
NC, NS = 2, 16
NW = NC * NS
D = 64
BAG = 8
C = 16
IPC = C * BAG
ROWS_PER_DMA = min(IPC, 128)
NDMA = IPC // ROWS_PER_DMA

B_POI = 1024 * 20
B_NB = 1024 * 20 * 8
TOTAL_BAGS = B_POI + B_NB
BAGS_PER_W = TOTAL_BAGS // NW
NCHUNKS = BAGS_PER_W // C
POI_GCHUNKS = B_POI // C
DEPTH = 3


def _sc_body(emb_hbm, poi_hbm, nb_hbm, opoi_hbm, onb_hbm, *scratch):
    wid = lax.axis_index("s") * NC + lax.axis_index("c")
    idx = scratch[0:DEPTH]
    rows = scratch[DEPTH:2 * DEPTH]
    outb = scratch[2 * DEPTH:3 * DEPTH]
    semg = scratch[3 * DEPTH:4 * DEPTH]
    semi = scratch[4 * DEPTH:5 * DEPTH]
    semo = scratch[5 * DEPTH:6 * DEPTH]
    lanes_hi = lax.iota(jnp.int32, L) >= BAG

    def fire_ids(cc, par):
        g = wid * NCHUNKS + cc

        @pl.when(g < POI_GCHUNKS)
        def _():
            pltpu.async_copy(poi_hbm.at[pl.ds(g * IPC, IPC)], idx[par],
                             semi[par])

        @pl.when(g >= POI_GCHUNKS)
        def _():
            pltpu.async_copy(nb_hbm.at[pl.ds((g - POI_GCHUNKS) * IPC, IPC)],
                             idx[par], semi[par])

    def wait_ids(par):
        pltpu.make_async_copy(poi_hbm.at[pl.ds(0, IPC)], idx[par],
                              semi[par]).wait()

    def fire_gathers(par):
        for i in range(NDMA):
            pltpu.async_copy(
                emb_hbm.at[idx[par].at[pl.ds(i * ROWS_PER_DMA,
                                             ROWS_PER_DMA)]],
                rows[par].at[pl.ds(i * ROWS_PER_DMA, ROWS_PER_DMA), :],
                semg[par])

    def wait_gathers(par):
        for i in range(NDMA):
            pltpu.make_async_copy(
                emb_hbm.at[idx[par].at[pl.ds(i * ROWS_PER_DMA,
                                             ROWS_PER_DMA)]],
                rows[par].at[pl.ds(i * ROWS_PER_DMA, ROWS_PER_DMA), :],
                semg[par]).wait()

    def fire_out(cc, par):
        g = wid * NCHUNKS + cc

        @pl.when(g < POI_GCHUNKS)
        def _():
            pltpu.async_copy(outb[par], opoi_hbm.at[pl.ds(g * C * D, C * D)],
                             semo[par])

        @pl.when(g >= POI_GCHUNKS)
        def _():
            pltpu.async_copy(
                outb[par],
                onb_hbm.at[pl.ds((g - POI_GCHUNKS) * C * D, C * D)],
                semo[par])

    def wait_out(par):
        pltpu.make_async_copy(outb[par], opoi_hbm.at[pl.ds(0, C * D)],
                              semo[par]).wait()

    hi_mask = jnp.int32(-65536)

    def compute(par):
        for t in range(C // 2):
            v = idx[par][pl.ds(t * L, L)]
            m = jnp.where(v != 0, 1.0, 0.0)
            cs = plsc.cumsum(m)
            c0 = jnp.broadcast_to(cs[BAG - 1], (L,))
            cnts = cs - jnp.where(lanes_hi, c0, 0.0)
            recv = 1.0 / jnp.maximum(cnts, 1.0)
            recs = (jnp.broadcast_to(recv[BAG - 1], (L,)),
                    jnp.broadcast_to(recv[2 * BAG - 1], (L,)))
            for h in range(2):
                b = t * 2 + h
                rbase = b * BAG
                for k in range(2):
                    x = plsc.bitcast(rows[par][rbase, pl.ds(k * 32, 32)],
                                     jnp.int32)
                    acc_lo = lax.shift_left(x, 16)
                    acc_hi = jnp.bitwise_and(x, hi_mask)
                    acc_lo = plsc.bitcast(acc_lo, jnp.float32)
                    acc_hi = plsc.bitcast(acc_hi, jnp.float32)
                    for j in range(1, BAG):
                        x = plsc.bitcast(
                            rows[par][rbase + j, pl.ds(k * 32, 32)],
                            jnp.int32)
                        acc_lo = acc_lo + plsc.bitcast(
                            lax.shift_left(x, 16), jnp.float32)
                        acc_hi = acc_hi + plsc.bitcast(
                            jnp.bitwise_and(x, hi_mask), jnp.float32)
                    outb[par][pl.ds(b * D + k * 32, L)] = acc_lo * recs[h]
                    outb[par][pl.ds(b * D + k * 32 + L, L)] = (
                        acc_hi * recs[h])

    for s in range(DEPTH):
        fire_ids(s, s)
    for s in range(DEPTH - 1):
        wait_ids(s)
        fire_gathers(s)

    def step(c2, carry):
        for par in range(DEPTH):
            c = c2 * DEPTH + par
            wait_gathers(par)

            @pl.when(c + DEPTH - 1 < NCHUNKS)
            def _():
                wait_ids((par + DEPTH - 1) % DEPTH)
                fire_gathers((par + DEPTH - 1) % DEPTH)

            @pl.when(c >= DEPTH)
            def _():
                wait_out(par)

            compute(par)

            @pl.when(c + DEPTH < NCHUNKS)
            def _():
                fire_ids(c + DEPTH, par)

            fire_out(c, par)
        return carry

    lax.fori_loop(0, NCHUNKS // DEPTH, step, 0)
    for s in range(DEPTH):
        wait_out(s)


@functools.partial(jax.jit, static_argnames=())
def _sc_pool(emb, poi_flat, nb_flat):
    kfn = pl.kernel(
        _sc_body,
        out_type=(jax.ShapeDtypeStruct((B_POI * D,), jnp.float32),
                  jax.ShapeDtypeStruct((B_NB * D,), jnp.float32)),
        mesh=plsc.VectorSubcoreMesh(core_axis_name="c", subcore_axis_name="s"),
        scratch_types=(
            [pltpu.VMEM((IPC,), jnp.int32) for _ in range(DEPTH)]
            + [pltpu.VMEM((IPC, D), jnp.bfloat16) for _ in range(DEPTH)]
            + [pltpu.VMEM((C * D,), jnp.float32) for _ in range(DEPTH)]
            + [pltpu.SemaphoreType.DMA for _ in range(3 * DEPTH)]
        ),
        compiler_params=pltpu.CompilerParams(use_tc_tiling_on_sc=False,
                                             needs_layout_passes=False),
    )
    return kfn(emb, poi_flat, nb_flat)


_COL_PERM = [blk * 32 + off
             for blk in range(2)
             for i in range(16)
             for off in (i, 16 + i)]


def kernel(poi_ids, neighbor_ids, embedding):
    poi_flat = poi_ids.reshape(-1).astype(jnp.int32)
    nb_flat = neighbor_ids.reshape(-1).astype(jnp.int32)
    emb_bf16 = embedding[:, jnp.array(_COL_PERM)].astype(jnp.bfloat16)
    opoi, onb = _sc_pool(emb_bf16, poi_flat, nb_flat)
    return (opoi.reshape(1024, 20, D), onb.reshape(1024, 20, BAG, D))
